# baseline (speedup 1.0000x reference)
import functools

import jax
import jax.numpy as jnp
from jax import lax
from jax.experimental import pallas as pl
from jax.experimental.pallas import tpu as pltpu
from jax.experimental.pallas import tpu_sc as plsc

N = 10000
E = 320000
D = 128

NC = 2
NS = 16
NW = NC * NS
EPW = E // NW
CHUNK = 80
NCHUNK = EPW // CHUNK
NSLICE = 5
SE = E // NSLICE
SEPW = SE // NW
PCH = 200
NPCH = SEPW // PCH
DCH = 2000
NDCH = EPW // DCH

_mesh = plsc.VectorSubcoreMesh(core_axis_name="c", subcore_axis_name="s")


def _ids():
    c = lax.axis_index("c")
    s = lax.axis_index("s")
    wid = s * NC + c
    return c, s, wid


def _deg_body(dst_hbm, ones_hbm, zeros_hbm, degp_hbm, idx_all, ones_v, accum, sem):
    c, s, wid = _ids()
    base = wid * EPW

    @pl.when(s == 0)
    def _():
        pltpu.sync_copy(zeros_hbm, accum)

    pltpu.sync_copy(dst_hbm.at[pl.ds(base, EPW)], idx_all)
    pltpu.sync_copy(ones_hbm, ones_v)
    plsc.subcore_barrier()

    def chunk(j, carry):
        off = pl.multiple_of(j * DCH, 8)
        pltpu.sync_copy(ones_v, accum.at[idx_all.at[pl.ds(off, DCH)]], add=True)
        return carry

    lax.fori_loop(0, NDCH, chunk, 0)
    plsc.subcore_barrier()

    @pl.when(s == 0)
    def _():
        pltpu.sync_copy(accum, degp_hbm.at[c])


_deg_kernel = pl.kernel(
    _deg_body,
    out_type=jax.ShapeDtypeStruct((NC, N), jnp.float32),
    mesh=_mesh,
    scratch_types=[
        pltpu.VMEM((EPW,), jnp.int32),
        pltpu.VMEM((DCH,), jnp.float32),
        pltpu.VMEM_SHARED((N,), jnp.float32),
        pltpu.SemaphoreType.DMA,
    ],
)


def _seg_body(src_hbm, dst_hbm, hs_hbm, zeros_hbm, segp_hbm,
              sidx_all, didx_all, rows0, rows1, accum,
              sem0, sem1, ssem0, ssem1):
    c, s, wid = _ids()
    base = wid * EPW

    @pl.when(s == 0)
    def _():
        pltpu.sync_copy(zeros_hbm, accum)

    pltpu.sync_copy(src_hbm.at[pl.ds(base, EPW)], sidx_all)
    pltpu.sync_copy(dst_hbm.at[pl.ds(base, EPW)], didx_all)
    plsc.subcore_barrier()

    def gstart(ch, rows, sem):
        o = pl.multiple_of(ch * CHUNK, 8)
        pltpu.async_copy(hs_hbm.at[sidx_all.at[pl.ds(o, CHUNK)]], rows, sem)

    def gwait(ch, rows, sem):
        o = pl.multiple_of(ch * CHUNK, 8)
        pltpu.make_async_copy(hs_hbm.at[sidx_all.at[pl.ds(o, CHUNK)]],
                              rows, sem).wait()

    def sstart(ch, rows, sem):
        o = pl.multiple_of(ch * CHUNK, 8)
        pltpu.async_copy(rows, accum.at[didx_all.at[pl.ds(o, CHUNK)]], sem,
                         add=True)

    def swait(ch, rows, sem):
        o = pl.multiple_of(ch * CHUNK, 8)
        pltpu.make_async_copy(rows, accum.at[didx_all.at[pl.ds(o, CHUNK)]],
                              sem).wait()

    gstart(0, rows0, sem0)

    def pairstep(t, carry):
        c0, c1, c2 = 2 * t, 2 * t + 1, 2 * t + 2
        gstart(c1, rows1, sem1)
        gwait(c0, rows0, sem0)
        sstart(c0, rows0, ssem0)
        gwait(c1, rows1, sem1)
        sstart(c1, rows1, ssem1)
        swait(c0, rows0, ssem0)
        gstart(c2, rows0, sem0)
        swait(c1, rows1, ssem1)
        return carry

    lax.fori_loop(0, (NCHUNK - 1) // 2, pairstep, 0)
    gwait(NCHUNK - 1, rows0, sem0)
    sstart(NCHUNK - 1, rows0, ssem0)
    swait(NCHUNK - 1, rows0, ssem0)
    plsc.subcore_barrier()

    @pl.when(s == 0)
    def _():
        pltpu.sync_copy(accum, segp_hbm.at[c])


_seg_kernel = pl.kernel(
    _seg_body,
    out_type=jax.ShapeDtypeStruct((NC, N, D), jnp.float32),
    mesh=_mesh,
    scratch_types=[
        pltpu.VMEM((EPW,), jnp.int32),
        pltpu.VMEM((EPW,), jnp.int32),
        pltpu.VMEM((CHUNK, D), jnp.float32),
        pltpu.VMEM((CHUNK, D), jnp.float32),
        pltpu.VMEM_SHARED((N, D), jnp.float32),
        pltpu.SemaphoreType.DMA,
        pltpu.SemaphoreType.DMA,
        pltpu.SemaphoreType.DMA,
        pltpu.SemaphoreType.DMA,
    ],
)


def _pair_body(src_hbm, dst_hbm, tab_hbm, xs_hbm, ys_hbm,
               sidx_all, didx_all, xr0, yr0, xr1, yr1,
               sx0, sy0, sx1, sy1, wx0, wy0, wx1, wy1):
    c, s, wid = _ids()
    base = wid * SEPW

    pltpu.sync_copy(src_hbm.at[pl.ds(base, SEPW)], sidx_all)
    pltpu.sync_copy(dst_hbm.at[pl.ds(base, SEPW)], didx_all)

    def gstart(ch, xr, yr, sx, sy):
        o = pl.multiple_of(ch * PCH, 8)
        pltpu.async_copy(tab_hbm.at[sidx_all.at[pl.ds(o, PCH)]], xr, sx)
        pltpu.async_copy(tab_hbm.at[didx_all.at[pl.ds(o, PCH)]], yr, sy)

    def gwait(ch, xr, yr, sx, sy):
        o = pl.multiple_of(ch * PCH, 8)
        pltpu.make_async_copy(tab_hbm.at[sidx_all.at[pl.ds(o, PCH)]], xr, sx).wait()
        pltpu.make_async_copy(tab_hbm.at[didx_all.at[pl.ds(o, PCH)]], yr, sy).wait()

    def wstart(ch, xr, yr, wx, wy):
        go = pl.multiple_of(base + ch * PCH, 8)
        pltpu.async_copy(xr, xs_hbm.at[pl.ds(go, PCH)], wx)
        pltpu.async_copy(yr, ys_hbm.at[pl.ds(go, PCH)], wy)

    def wwait(ch, xr, yr, wx, wy):
        go = pl.multiple_of(base + ch * PCH, 8)
        pltpu.make_async_copy(xr, xs_hbm.at[pl.ds(go, PCH)], wx).wait()
        pltpu.make_async_copy(yr, ys_hbm.at[pl.ds(go, PCH)], wy).wait()

    gstart(0, xr0, yr0, sx0, sy0)

    def pairstep(t, carry):
        c0, c1, c2 = 2 * t, 2 * t + 1, 2 * t + 2
        gstart(c1, xr1, yr1, sx1, sy1)
        gwait(c0, xr0, yr0, sx0, sy0)
        wstart(c0, xr0, yr0, wx0, wy0)
        gwait(c1, xr1, yr1, sx1, sy1)
        wstart(c1, xr1, yr1, wx1, wy1)
        wwait(c0, xr0, yr0, wx0, wy0)
        gstart(c2, xr0, yr0, sx0, sy0)
        wwait(c1, xr1, yr1, wx1, wy1)
        return carry

    lax.fori_loop(0, NPCH // 2 - 1, pairstep, 0)
    c0, c1 = NPCH - 2, NPCH - 1
    gstart(c1, xr1, yr1, sx1, sy1)
    gwait(c0, xr0, yr0, sx0, sy0)
    wstart(c0, xr0, yr0, wx0, wy0)
    gwait(c1, xr1, yr1, sx1, sy1)
    wstart(c1, xr1, yr1, wx1, wy1)
    wwait(c0, xr0, yr0, wx0, wy0)
    wwait(c1, xr1, yr1, wx1, wy1)


_pair_kernel = pl.kernel(
    _pair_body,
    out_type=(jax.ShapeDtypeStruct((SE, D), jnp.float32),
              jax.ShapeDtypeStruct((SE, D), jnp.float32)),
    mesh=_mesh,
    scratch_types=[
        pltpu.VMEM((SEPW,), jnp.int32),
        pltpu.VMEM((SEPW,), jnp.int32),
        pltpu.VMEM((PCH, D), jnp.float32),
        pltpu.VMEM((PCH, D), jnp.float32),
        pltpu.VMEM((PCH, D), jnp.float32),
        pltpu.VMEM((PCH, D), jnp.float32),
        pltpu.SemaphoreType.DMA,
        pltpu.SemaphoreType.DMA,
        pltpu.SemaphoreType.DMA,
        pltpu.SemaphoreType.DMA,
        pltpu.SemaphoreType.DMA,
        pltpu.SemaphoreType.DMA,
        pltpu.SemaphoreType.DMA,
        pltpu.SemaphoreType.DMA,
    ],
)


NBLK = 1000
EBLK = 2000


def _dinv_block(degp):
    deg = 1.0 + degp[0] + degp[1]
    return lax.rsqrt(deg)


def _tc1_body(x_ref, w_ref, degp_ref, hs_ref):
    h = jnp.dot(x_ref[...], w_ref[...], preferred_element_type=jnp.float32)
    hs_ref[...] = _dinv_block(degp_ref[...]) * h


def _tc2_body(segp_ref, hs_ref, degp_ref, b_ref, w_ref, out_ref):
    dinv = _dinv_block(degp_ref[...])
    seg = segp_ref[0] + segp_ref[1] + hs_ref[...]
    o = jax.nn.relu(dinv * seg + b_ref[...])
    h = jnp.dot(o, w_ref[...], preferred_element_type=jnp.float32)
    out_ref[...] = dinv * h


def _tc3_body(segp_ref, hs_ref, degp_ref, b_ref, out_ref):
    dinv = _dinv_block(degp_ref[...])
    seg = segp_ref[0] + segp_ref[1] + hs_ref[...]
    out_ref[...] = jax.nn.relu(dinv * seg + b_ref[...])


def _head_body(xs_ref, ys_ref, fa_ref, fb_ref, b1_ref, w2_ref, b2_ref, out_ref):
    x = xs_ref[...]
    y = ys_ref[...]
    h = jnp.dot(x * y, fa_ref[...], preferred_element_type=jnp.float32)
    h += jnp.dot(x - y, fb_ref[...], preferred_element_type=jnp.float32)
    h = jax.nn.relu(h + b1_ref[...])
    p = jnp.dot(h, w2_ref[...], preferred_element_type=jnp.float32)
    out_ref[...] = jax.nn.sigmoid(p + b2_ref[...])


def _node_spec(d):
    return pl.BlockSpec((NBLK, d), lambda i: (i, 0))


_full = lambda shape: pl.BlockSpec(shape, lambda i: tuple(0 for _ in shape))
_degp_spec = pl.BlockSpec((NC, NBLK, 1), lambda i: (0, i, 0))
_segp_spec = pl.BlockSpec((NC, NBLK, D), lambda i: (0, i, 0))

_tc1 = pl.pallas_call(
    _tc1_body,
    grid=(N // NBLK,),
    in_specs=[_node_spec(D), _full((D, D)), _degp_spec],
    out_specs=_node_spec(D),
    out_shape=jax.ShapeDtypeStruct((N, D), jnp.float32),
)

_tc2 = pl.pallas_call(
    _tc2_body,
    grid=(N // NBLK,),
    in_specs=[_segp_spec, _node_spec(D), _degp_spec, _full((1, D)), _full((D, D))],
    out_specs=_node_spec(D),
    out_shape=jax.ShapeDtypeStruct((N, D), jnp.float32),
)

_tc3 = pl.pallas_call(
    _tc3_body,
    grid=(N // NBLK,),
    in_specs=[_segp_spec, _node_spec(D), _degp_spec, _full((1, D))],
    out_specs=_node_spec(D),
    out_shape=jax.ShapeDtypeStruct((N, D), jnp.float32),
)

_head = pl.pallas_call(
    _head_body,
    grid=(SE // EBLK,),
    in_specs=[
        pl.BlockSpec((EBLK, D), lambda i: (i, 0)),
        pl.BlockSpec((EBLK, D), lambda i: (i, 0)),
        _full((D, D)), _full((D, D)), _full((1, D)),
        _full((D, 1)), _full((1, 1)),
    ],
    out_specs=pl.BlockSpec((EBLK, 1), lambda i: (i, 0)),
    out_shape=jax.ShapeDtypeStruct((SE, 1), jnp.float32),
)


def kernel(node_features, W1, b1, W2, b2, fc1_w, fc1_b, fc2_w, fc2_b, edge_index):
    src = edge_index[0].astype(jnp.int32)
    dst = edge_index[1].astype(jnp.int32)
    ones1 = jnp.ones((DCH,), jnp.float32)
    zeros1 = jnp.zeros((N,), jnp.float32)
    zerosD = jnp.zeros((N, D), jnp.float32)
    b1r = b1.reshape(1, D)
    b2r = b2.reshape(1, D)
    fc1_br = fc1_b.reshape(1, D)
    fc2_br = fc2_b.reshape(1, 1)
    fa = fc1_w[:D]
    fb = fc1_w[D:]

    degp = _deg_kernel(dst, ones1, zeros1).reshape(NC, N, 1)
    hs1 = _tc1(node_features, W1, degp)
    segp1 = _seg_kernel(src, dst, hs1, zerosD)
    hs2 = _tc2(segp1, hs1, degp, b1r, W2)
    segp2 = _seg_kernel(src, dst, hs2, zerosD)
    out2 = _tc3(segp2, hs2, degp, b2r)
    probs = []
    for i in range(NSLICE):
        xs, ys = _pair_kernel(src[i * SE:(i + 1) * SE],
                              dst[i * SE:(i + 1) * SE], out2)
        probs.append(_head(xs, ys, fa, fb, fc1_br, fc2_w, fc2_br))
    return jnp.concatenate(probs, axis=0)


# speedup vs baseline: 1.0888x; 1.0888x over previous
import functools

import jax
import jax.numpy as jnp
from jax import lax
from jax.experimental import pallas as pl
from jax.experimental.pallas import tpu as pltpu
from jax.experimental.pallas import tpu_sc as plsc

N = 10000
E = 320000
D = 128

NC = 2
NS = 16
NW = NC * NS
EPW = E // NW
CHUNK = 80
NCHUNK = EPW // CHUNK
NSLICE = 5
SE = E // NSLICE
SEPW = SE // NW
PCH = 200
NPCH = SEPW // PCH
DCH = 2000
NDCH = EPW // DCH

_mesh = plsc.VectorSubcoreMesh(core_axis_name="c", subcore_axis_name="s")


def _ids():
    c = lax.axis_index("c")
    s = lax.axis_index("s")
    wid = s * NC + c
    return c, s, wid


def _deg_body(dst_hbm, ones_hbm, zeros_hbm, degp_hbm, idx_all, ones_v, accum, sem):
    c, s, wid = _ids()
    base = wid * EPW

    @pl.when(s == 0)
    def _():
        pltpu.sync_copy(zeros_hbm, accum)

    pltpu.sync_copy(dst_hbm.at[pl.ds(base, EPW)], idx_all)
    pltpu.sync_copy(ones_hbm, ones_v)
    plsc.subcore_barrier()

    def chunk(j, carry):
        off = pl.multiple_of(j * DCH, 8)
        pltpu.sync_copy(ones_v, accum.at[idx_all.at[pl.ds(off, DCH)]], add=True)
        return carry

    lax.fori_loop(0, NDCH, chunk, 0)
    plsc.subcore_barrier()

    @pl.when(s == 0)
    def _():
        pltpu.sync_copy(accum, degp_hbm.at[c])


_deg_kernel = pl.kernel(
    _deg_body,
    out_type=jax.ShapeDtypeStruct((NC, N), jnp.float32),
    mesh=_mesh,
    scratch_types=[
        pltpu.VMEM((EPW,), jnp.int32),
        pltpu.VMEM((DCH,), jnp.float32),
        pltpu.VMEM_SHARED((N,), jnp.float32),
        pltpu.SemaphoreType.DMA,
    ],
)


ZR = 624
ZR_LAST = N - 15 * ZR


def _seg_body(src_hbm, dst_hbm, hs_hbm, zeros_hbm, segp_hbm,
              sidx_all, didx_all, rows0, rows1, accum, sem0, sem1):
    c, s, wid = _ids()
    base = wid * EPW

    @pl.when(s < 15)
    def _():
        o = pl.multiple_of(s * ZR, 8)
        pltpu.sync_copy(zeros_hbm.at[pl.ds(o, ZR)], accum.at[pl.ds(o, ZR)])

    @pl.when(s == 15)
    def _():
        pltpu.sync_copy(zeros_hbm.at[pl.ds(15 * ZR, ZR_LAST)],
                        accum.at[pl.ds(15 * ZR, ZR_LAST)])

    pltpu.sync_copy(src_hbm.at[pl.ds(base, EPW)], sidx_all)
    pltpu.sync_copy(dst_hbm.at[pl.ds(base, EPW)], didx_all)
    plsc.subcore_barrier()

    def gstart(ch, rows, sem):
        o = pl.multiple_of(ch * CHUNK, 8)
        pltpu.async_copy(hs_hbm.at[sidx_all.at[pl.ds(o, CHUNK)]], rows, sem)

    def gwait(ch, rows, sem):
        o = pl.multiple_of(ch * CHUNK, 8)
        pltpu.make_async_copy(hs_hbm.at[sidx_all.at[pl.ds(o, CHUNK)]],
                              rows, sem).wait()

    def scat(ch, rows):
        o = pl.multiple_of(ch * CHUNK, 8)
        pltpu.sync_copy(rows, accum.at[didx_all.at[pl.ds(o, CHUNK)]], add=True)

    gstart(0, rows0, sem0)

    def pairstep(t, carry):
        c0, c1, c2 = 2 * t, 2 * t + 1, 2 * t + 2
        gstart(c1, rows1, sem1)
        gwait(c0, rows0, sem0)
        scat(c0, rows0)
        gstart(c2, rows0, sem0)
        gwait(c1, rows1, sem1)
        scat(c1, rows1)
        return carry

    lax.fori_loop(0, (NCHUNK - 1) // 2, pairstep, 0)
    gwait(NCHUNK - 1, rows0, sem0)
    scat(NCHUNK - 1, rows0)
    plsc.subcore_barrier()

    @pl.when(s < 15)
    def _():
        o = pl.multiple_of(s * ZR, 8)
        pltpu.sync_copy(accum.at[pl.ds(o, ZR)], segp_hbm.at[c, pl.ds(o, ZR)])

    @pl.when(s == 15)
    def _():
        pltpu.sync_copy(accum.at[pl.ds(15 * ZR, ZR_LAST)],
                        segp_hbm.at[c, pl.ds(15 * ZR, ZR_LAST)])


_seg_kernel = pl.kernel(
    _seg_body,
    out_type=jax.ShapeDtypeStruct((NC, N, D), jnp.float32),
    mesh=_mesh,
    scratch_types=[
        pltpu.VMEM((EPW,), jnp.int32),
        pltpu.VMEM((EPW,), jnp.int32),
        pltpu.VMEM((CHUNK, D), jnp.float32),
        pltpu.VMEM((CHUNK, D), jnp.float32),
        pltpu.VMEM_SHARED((N, D), jnp.float32),
        pltpu.SemaphoreType.DMA,
        pltpu.SemaphoreType.DMA,
    ],
)


def _pair_body(src_hbm, dst_hbm, tab_hbm, xs_hbm, ys_hbm,
               sidx_all, didx_all, xr0, yr0, xr1, yr1,
               sx0, sy0, sx1, sy1):
    c, s, wid = _ids()
    base = wid * SEPW

    pltpu.sync_copy(src_hbm.at[pl.ds(base, SEPW)], sidx_all)
    pltpu.sync_copy(dst_hbm.at[pl.ds(base, SEPW)], didx_all)

    def gstart(ch, xr, yr, sx, sy):
        o = pl.multiple_of(ch * PCH, 8)
        pltpu.async_copy(tab_hbm.at[sidx_all.at[pl.ds(o, PCH)]], xr, sx)
        pltpu.async_copy(tab_hbm.at[didx_all.at[pl.ds(o, PCH)]], yr, sy)

    def finish(ch, xr, yr, sx, sy):
        o = pl.multiple_of(ch * PCH, 8)
        go = pl.multiple_of(base + ch * PCH, 8)
        pltpu.make_async_copy(tab_hbm.at[sidx_all.at[pl.ds(o, PCH)]], xr, sx).wait()
        pltpu.make_async_copy(tab_hbm.at[didx_all.at[pl.ds(o, PCH)]], yr, sy).wait()
        pltpu.sync_copy(xr, xs_hbm.at[pl.ds(go, PCH)])
        pltpu.sync_copy(yr, ys_hbm.at[pl.ds(go, PCH)])

    gstart(0, xr0, yr0, sx0, sy0)

    def pairstep(t, carry):
        c0, c1, c2 = 2 * t, 2 * t + 1, 2 * t + 2
        gstart(c1, xr1, yr1, sx1, sy1)
        finish(c0, xr0, yr0, sx0, sy0)
        gstart(c2, xr0, yr0, sx0, sy0)
        finish(c1, xr1, yr1, sx1, sy1)
        return carry

    lax.fori_loop(0, NPCH // 2 - 1, pairstep, 0)
    gstart(NPCH - 1, xr1, yr1, sx1, sy1)
    finish(NPCH - 2, xr0, yr0, sx0, sy0)
    finish(NPCH - 1, xr1, yr1, sx1, sy1)


_pair_kernel = pl.kernel(
    _pair_body,
    out_type=(jax.ShapeDtypeStruct((SE, D), jnp.float32),
              jax.ShapeDtypeStruct((SE, D), jnp.float32)),
    mesh=_mesh,
    scratch_types=[
        pltpu.VMEM((SEPW,), jnp.int32),
        pltpu.VMEM((SEPW,), jnp.int32),
        pltpu.VMEM((PCH, D), jnp.float32),
        pltpu.VMEM((PCH, D), jnp.float32),
        pltpu.VMEM((PCH, D), jnp.float32),
        pltpu.VMEM((PCH, D), jnp.float32),
        pltpu.SemaphoreType.DMA,
        pltpu.SemaphoreType.DMA,
        pltpu.SemaphoreType.DMA,
        pltpu.SemaphoreType.DMA,
    ],
)


NBLK = 1000
EBLK = 2000


def _dinv_block(degp):
    deg = 1.0 + degp[0] + degp[1]
    return lax.rsqrt(deg)


def _tc1_body(x_ref, w_ref, degp_ref, hs_ref):
    h = jnp.dot(x_ref[...], w_ref[...], preferred_element_type=jnp.float32)
    hs_ref[...] = _dinv_block(degp_ref[...]) * h


def _tc2_body(segp_ref, hs_ref, degp_ref, b_ref, w_ref, out_ref):
    dinv = _dinv_block(degp_ref[...])
    seg = segp_ref[0] + segp_ref[1] + hs_ref[...]
    o = jax.nn.relu(dinv * seg + b_ref[...])
    h = jnp.dot(o, w_ref[...], preferred_element_type=jnp.float32)
    out_ref[...] = dinv * h


def _tc3_body(segp_ref, hs_ref, degp_ref, b_ref, out_ref):
    dinv = _dinv_block(degp_ref[...])
    seg = segp_ref[0] + segp_ref[1] + hs_ref[...]
    out_ref[...] = jax.nn.relu(dinv * seg + b_ref[...])


def _head_body(xs_ref, ys_ref, fa_ref, fb_ref, b1_ref, w2_ref, b2_ref, out_ref):
    x = xs_ref[...]
    y = ys_ref[...]
    h = jnp.dot(x * y, fa_ref[...], preferred_element_type=jnp.float32)
    h += jnp.dot(x - y, fb_ref[...], preferred_element_type=jnp.float32)
    h = jax.nn.relu(h + b1_ref[...])
    p = jnp.dot(h, w2_ref[...], preferred_element_type=jnp.float32)
    out_ref[...] = jax.nn.sigmoid(p + b2_ref[...])


def _node_spec(d):
    return pl.BlockSpec((NBLK, d), lambda i: (i, 0))


_full = lambda shape: pl.BlockSpec(shape, lambda i: tuple(0 for _ in shape))
_degp_spec = pl.BlockSpec((NC, NBLK, 1), lambda i: (0, i, 0))
_segp_spec = pl.BlockSpec((NC, NBLK, D), lambda i: (0, i, 0))

_tc1 = pl.pallas_call(
    _tc1_body,
    grid=(N // NBLK,),
    in_specs=[_node_spec(D), _full((D, D)), _degp_spec],
    out_specs=_node_spec(D),
    out_shape=jax.ShapeDtypeStruct((N, D), jnp.float32),
)

_tc2 = pl.pallas_call(
    _tc2_body,
    grid=(N // NBLK,),
    in_specs=[_segp_spec, _node_spec(D), _degp_spec, _full((1, D)), _full((D, D))],
    out_specs=_node_spec(D),
    out_shape=jax.ShapeDtypeStruct((N, D), jnp.float32),
)

_tc3 = pl.pallas_call(
    _tc3_body,
    grid=(N // NBLK,),
    in_specs=[_segp_spec, _node_spec(D), _degp_spec, _full((1, D))],
    out_specs=_node_spec(D),
    out_shape=jax.ShapeDtypeStruct((N, D), jnp.float32),
)

_head = pl.pallas_call(
    _head_body,
    grid=(SE // EBLK,),
    in_specs=[
        pl.BlockSpec((EBLK, D), lambda i: (i, 0)),
        pl.BlockSpec((EBLK, D), lambda i: (i, 0)),
        _full((D, D)), _full((D, D)), _full((1, D)),
        _full((D, 1)), _full((1, 1)),
    ],
    out_specs=pl.BlockSpec((EBLK, 1), lambda i: (i, 0)),
    out_shape=jax.ShapeDtypeStruct((SE, 1), jnp.float32),
)


def kernel(node_features, W1, b1, W2, b2, fc1_w, fc1_b, fc2_w, fc2_b, edge_index):
    src = edge_index[0].astype(jnp.int32)
    dst = edge_index[1].astype(jnp.int32)
    ones1 = jnp.ones((DCH,), jnp.float32)
    zeros1 = jnp.zeros((N,), jnp.float32)
    zerosD = jnp.zeros((N, D), jnp.float32)
    b1r = b1.reshape(1, D)
    b2r = b2.reshape(1, D)
    fc1_br = fc1_b.reshape(1, D)
    fc2_br = fc2_b.reshape(1, 1)
    fa = fc1_w[:D]
    fb = fc1_w[D:]

    degp = _deg_kernel(dst, ones1, zeros1).reshape(NC, N, 1)
    hs1 = _tc1(node_features, W1, degp)
    segp1 = _seg_kernel(src, dst, hs1, zerosD)
    hs2 = _tc2(segp1, hs1, degp, b1r, W2)
    segp2 = _seg_kernel(src, dst, hs2, zerosD)
    out2 = _tc3(segp2, hs2, degp, b2r)
    probs = []
    for i in range(NSLICE):
        xs, ys = _pair_kernel(src[i * SE:(i + 1) * SE],
                              dst[i * SE:(i + 1) * SE], out2)
        probs.append(_head(xs, ys, fa, fb, fc1_br, fc2_w, fc2_br))
    return jnp.concatenate(probs, axis=0)


# speedup vs baseline: 1.1116x; 1.0210x over previous
import functools

import jax
import jax.numpy as jnp
from jax import lax
from jax.experimental import pallas as pl
from jax.experimental.pallas import tpu as pltpu
from jax.experimental.pallas import tpu_sc as plsc

N = 10000
E = 320000
D = 128

NC = 2
NS = 16
NW = NC * NS
EPW = E // NW
CHUNK = 112
NCHUNK = 89
TAIL = EPW - NCHUNK * CHUNK
NSLICE = 5
SE = E // NSLICE
SEPW = SE // NW
PCH = 200
NPCH = SEPW // PCH
DCH = 2000
NDCH = EPW // DCH

_mesh = plsc.VectorSubcoreMesh(core_axis_name="c", subcore_axis_name="s")


def _ids():
    c = lax.axis_index("c")
    s = lax.axis_index("s")
    wid = s * NC + c
    return c, s, wid


def _deg_body(dst_hbm, ones_hbm, zeros_hbm, degp_hbm, idx_all, ones_v, accum, sem):
    c, s, wid = _ids()
    base = wid * EPW

    @pl.when(s == 0)
    def _():
        pltpu.sync_copy(zeros_hbm, accum)

    pltpu.sync_copy(dst_hbm.at[pl.ds(base, EPW)], idx_all)
    pltpu.sync_copy(ones_hbm, ones_v)
    plsc.subcore_barrier()

    def chunk(j, carry):
        off = pl.multiple_of(j * DCH, 8)
        pltpu.sync_copy(ones_v, accum.at[idx_all.at[pl.ds(off, DCH)]], add=True)
        return carry

    lax.fori_loop(0, NDCH, chunk, 0)
    plsc.subcore_barrier()

    @pl.when(s == 0)
    def _():
        pltpu.sync_copy(accum, degp_hbm.at[c])


_deg_kernel = pl.kernel(
    _deg_body,
    out_type=jax.ShapeDtypeStruct((NC, N), jnp.float32),
    mesh=_mesh,
    scratch_types=[
        pltpu.VMEM((EPW,), jnp.int32),
        pltpu.VMEM((DCH,), jnp.float32),
        pltpu.VMEM_SHARED((N,), jnp.float32),
        pltpu.SemaphoreType.DMA,
    ],
)


ZR = 624
ZR_LAST = N - 15 * ZR


def _seg_body(src_hbm, dst_hbm, hs_hbm, zeros_hbm, segp_hbm,
              sidx_all, didx_all, rows0, rows1, accum, sem0, sem1):
    c, s, wid = _ids()
    base = wid * EPW

    @pl.when(s < 15)
    def _():
        o = pl.multiple_of(s * ZR, 8)
        pltpu.sync_copy(zeros_hbm.at[pl.ds(o, ZR)], accum.at[pl.ds(o, ZR)])

    @pl.when(s == 15)
    def _():
        pltpu.sync_copy(zeros_hbm.at[pl.ds(15 * ZR, ZR_LAST)],
                        accum.at[pl.ds(15 * ZR, ZR_LAST)])

    pltpu.sync_copy(src_hbm.at[pl.ds(base, EPW)], sidx_all)
    pltpu.sync_copy(dst_hbm.at[pl.ds(base, EPW)], didx_all)
    plsc.subcore_barrier()

    def gstart(ch, rows, sem):
        o = pl.multiple_of(ch * CHUNK, 8)
        pltpu.async_copy(hs_hbm.at[sidx_all.at[pl.ds(o, CHUNK)]], rows, sem)

    def gwait(ch, rows, sem):
        o = pl.multiple_of(ch * CHUNK, 8)
        pltpu.make_async_copy(hs_hbm.at[sidx_all.at[pl.ds(o, CHUNK)]],
                              rows, sem).wait()

    def scat(ch, rows):
        o = pl.multiple_of(ch * CHUNK, 8)
        pltpu.sync_copy(rows, accum.at[didx_all.at[pl.ds(o, CHUNK)]], add=True)

    gstart(0, rows0, sem0)

    def pairstep(t, carry):
        c0, c1, c2 = 2 * t, 2 * t + 1, 2 * t + 2
        gstart(c1, rows1, sem1)
        gwait(c0, rows0, sem0)
        scat(c0, rows0)
        gstart(c2, rows0, sem0)
        gwait(c1, rows1, sem1)
        scat(c1, rows1)
        return carry

    lax.fori_loop(0, (NCHUNK - 1) // 2, pairstep, 0)
    gwait(NCHUNK - 1, rows0, sem0)
    scat(NCHUNK - 1, rows0)
    to = pl.multiple_of(NCHUNK * CHUNK, 8)
    tail_rows = rows1.at[pl.ds(0, TAIL)]
    pltpu.async_copy(hs_hbm.at[sidx_all.at[pl.ds(to, TAIL)]],
                     tail_rows, sem1).wait()
    pltpu.sync_copy(tail_rows, accum.at[didx_all.at[pl.ds(to, TAIL)]], add=True)
    plsc.subcore_barrier()

    @pl.when(s < 15)
    def _():
        o = pl.multiple_of(s * ZR, 8)
        pltpu.sync_copy(accum.at[pl.ds(o, ZR)], segp_hbm.at[c, pl.ds(o, ZR)])

    @pl.when(s == 15)
    def _():
        pltpu.sync_copy(accum.at[pl.ds(15 * ZR, ZR_LAST)],
                        segp_hbm.at[c, pl.ds(15 * ZR, ZR_LAST)])


_seg_kernel = pl.kernel(
    _seg_body,
    out_type=jax.ShapeDtypeStruct((NC, N, D), jnp.float32),
    mesh=_mesh,
    scratch_types=[
        pltpu.VMEM((EPW,), jnp.int32),
        pltpu.VMEM((EPW,), jnp.int32),
        pltpu.VMEM((CHUNK, D), jnp.float32),
        pltpu.VMEM((CHUNK, D), jnp.float32),
        pltpu.VMEM_SHARED((N, D), jnp.float32),
        pltpu.SemaphoreType.DMA,
        pltpu.SemaphoreType.DMA,
    ],
)


def _pair_body(src_hbm, dst_hbm, tab_hbm, xs_hbm, ys_hbm,
               sidx_all, didx_all, xr0, yr0, xr1, yr1,
               sx0, sy0, sx1, sy1):
    c, s, wid = _ids()
    base = wid * SEPW

    pltpu.sync_copy(src_hbm.at[pl.ds(base, SEPW)], sidx_all)
    pltpu.sync_copy(dst_hbm.at[pl.ds(base, SEPW)], didx_all)

    def gstart(ch, xr, yr, sx, sy):
        o = pl.multiple_of(ch * PCH, 8)
        pltpu.async_copy(tab_hbm.at[sidx_all.at[pl.ds(o, PCH)]], xr, sx)
        pltpu.async_copy(tab_hbm.at[didx_all.at[pl.ds(o, PCH)]], yr, sy)

    def finish(ch, xr, yr, sx, sy):
        o = pl.multiple_of(ch * PCH, 8)
        go = pl.multiple_of(base + ch * PCH, 8)
        pltpu.make_async_copy(tab_hbm.at[sidx_all.at[pl.ds(o, PCH)]], xr, sx).wait()
        pltpu.make_async_copy(tab_hbm.at[didx_all.at[pl.ds(o, PCH)]], yr, sy).wait()
        pltpu.sync_copy(xr, xs_hbm.at[pl.ds(go, PCH)])
        pltpu.sync_copy(yr, ys_hbm.at[pl.ds(go, PCH)])

    gstart(0, xr0, yr0, sx0, sy0)

    def pairstep(t, carry):
        c0, c1, c2 = 2 * t, 2 * t + 1, 2 * t + 2
        gstart(c1, xr1, yr1, sx1, sy1)
        finish(c0, xr0, yr0, sx0, sy0)
        gstart(c2, xr0, yr0, sx0, sy0)
        finish(c1, xr1, yr1, sx1, sy1)
        return carry

    lax.fori_loop(0, NPCH // 2 - 1, pairstep, 0)
    gstart(NPCH - 1, xr1, yr1, sx1, sy1)
    finish(NPCH - 2, xr0, yr0, sx0, sy0)
    finish(NPCH - 1, xr1, yr1, sx1, sy1)


_pair_kernel = pl.kernel(
    _pair_body,
    out_type=(jax.ShapeDtypeStruct((SE, D), jnp.float32),
              jax.ShapeDtypeStruct((SE, D), jnp.float32)),
    mesh=_mesh,
    scratch_types=[
        pltpu.VMEM((SEPW,), jnp.int32),
        pltpu.VMEM((SEPW,), jnp.int32),
        pltpu.VMEM((PCH, D), jnp.float32),
        pltpu.VMEM((PCH, D), jnp.float32),
        pltpu.VMEM((PCH, D), jnp.float32),
        pltpu.VMEM((PCH, D), jnp.float32),
        pltpu.SemaphoreType.DMA,
        pltpu.SemaphoreType.DMA,
        pltpu.SemaphoreType.DMA,
        pltpu.SemaphoreType.DMA,
    ],
)


NBLK = 1000
EBLK = 2000


def _dinv_block(degp):
    deg = 1.0 + degp[0] + degp[1]
    return lax.rsqrt(deg)


def _tc1_body(x_ref, w_ref, degp_ref, hs_ref):
    h = jnp.dot(x_ref[...], w_ref[...], preferred_element_type=jnp.float32)
    hs_ref[...] = _dinv_block(degp_ref[...]) * h


def _tc2_body(segp_ref, hs_ref, degp_ref, b_ref, w_ref, out_ref):
    dinv = _dinv_block(degp_ref[...])
    seg = segp_ref[0] + segp_ref[1] + hs_ref[...]
    o = jax.nn.relu(dinv * seg + b_ref[...])
    h = jnp.dot(o, w_ref[...], preferred_element_type=jnp.float32)
    out_ref[...] = dinv * h


def _tc3_body(segp_ref, hs_ref, degp_ref, b_ref, out_ref):
    dinv = _dinv_block(degp_ref[...])
    seg = segp_ref[0] + segp_ref[1] + hs_ref[...]
    out_ref[...] = jax.nn.relu(dinv * seg + b_ref[...])


def _head_body(xs_ref, ys_ref, fa_ref, fb_ref, b1_ref, w2_ref, b2_ref, out_ref):
    x = xs_ref[...]
    y = ys_ref[...]
    h = jnp.dot(x * y, fa_ref[...], preferred_element_type=jnp.float32)
    h += jnp.dot(x - y, fb_ref[...], preferred_element_type=jnp.float32)
    h = jax.nn.relu(h + b1_ref[...])
    p = jnp.dot(h, w2_ref[...], preferred_element_type=jnp.float32)
    out_ref[...] = jax.nn.sigmoid(p + b2_ref[...])


def _node_spec(d):
    return pl.BlockSpec((NBLK, d), lambda i: (i, 0))


_full = lambda shape: pl.BlockSpec(shape, lambda i: tuple(0 for _ in shape))
_degp_spec = pl.BlockSpec((NC, NBLK, 1), lambda i: (0, i, 0))
_segp_spec = pl.BlockSpec((NC, NBLK, D), lambda i: (0, i, 0))

_tc1 = pl.pallas_call(
    _tc1_body,
    grid=(N // NBLK,),
    in_specs=[_node_spec(D), _full((D, D)), _degp_spec],
    out_specs=_node_spec(D),
    out_shape=jax.ShapeDtypeStruct((N, D), jnp.float32),
)

_tc2 = pl.pallas_call(
    _tc2_body,
    grid=(N // NBLK,),
    in_specs=[_segp_spec, _node_spec(D), _degp_spec, _full((1, D)), _full((D, D))],
    out_specs=_node_spec(D),
    out_shape=jax.ShapeDtypeStruct((N, D), jnp.float32),
)

_tc3 = pl.pallas_call(
    _tc3_body,
    grid=(N // NBLK,),
    in_specs=[_segp_spec, _node_spec(D), _degp_spec, _full((1, D))],
    out_specs=_node_spec(D),
    out_shape=jax.ShapeDtypeStruct((N, D), jnp.float32),
)

_head = pl.pallas_call(
    _head_body,
    grid=(SE // EBLK,),
    in_specs=[
        pl.BlockSpec((EBLK, D), lambda i: (i, 0)),
        pl.BlockSpec((EBLK, D), lambda i: (i, 0)),
        _full((D, D)), _full((D, D)), _full((1, D)),
        _full((D, 1)), _full((1, 1)),
    ],
    out_specs=pl.BlockSpec((EBLK, 1), lambda i: (i, 0)),
    out_shape=jax.ShapeDtypeStruct((SE, 1), jnp.float32),
)


def kernel(node_features, W1, b1, W2, b2, fc1_w, fc1_b, fc2_w, fc2_b, edge_index):
    src = edge_index[0].astype(jnp.int32)
    dst = edge_index[1].astype(jnp.int32)
    ones1 = jnp.ones((DCH,), jnp.float32)
    zeros1 = jnp.zeros((N,), jnp.float32)
    zerosD = jnp.zeros((N, D), jnp.float32)
    b1r = b1.reshape(1, D)
    b2r = b2.reshape(1, D)
    fc1_br = fc1_b.reshape(1, D)
    fc2_br = fc2_b.reshape(1, 1)
    fa = fc1_w[:D]
    fb = fc1_w[D:]

    degp = _deg_kernel(dst, ones1, zeros1).reshape(NC, N, 1)
    hs1 = _tc1(node_features, W1, degp)
    segp1 = _seg_kernel(src, dst, hs1, zerosD)
    hs2 = _tc2(segp1, hs1, degp, b1r, W2)
    segp2 = _seg_kernel(src, dst, hs2, zerosD)
    out2 = _tc3(segp2, hs2, degp, b2r)
    probs = []
    for i in range(NSLICE):
        xs, ys = _pair_kernel(src[i * SE:(i + 1) * SE],
                              dst[i * SE:(i + 1) * SE], out2)
        probs.append(_head(xs, ys, fa, fb, fc1_br, fc2_w, fc2_br))
    return jnp.concatenate(probs, axis=0)


# speedup vs baseline: 1.1544x; 1.0385x over previous
import functools

import jax
import jax.numpy as jnp
from jax import lax
from jax.experimental import pallas as pl
from jax.experimental.pallas import tpu as pltpu
from jax.experimental.pallas import tpu_sc as plsc

N = 10000
E = 320000
D = 128

NC = 2
NS = 16
NW = NC * NS
EPW = E // NW
CHUNK = 112
NCHUNK = 89
TAIL = EPW - NCHUNK * CHUNK
NSLICE = 5
SE = E // NSLICE
SEPW = SE // NW
PCH = 200
NPCH = SEPW // PCH
DCH = 2000
NDCH = EPW // DCH

_mesh = plsc.VectorSubcoreMesh(core_axis_name="c", subcore_axis_name="s")


def _ids():
    c = lax.axis_index("c")
    s = lax.axis_index("s")
    wid = s * NC + c
    return c, s, wid


def _deg_body(dst_hbm, ones_hbm, zeros_hbm, degp_hbm, idx_all, ones_v, accum, sem):
    c, s, wid = _ids()
    base = wid * EPW

    @pl.when(s == 0)
    def _():
        pltpu.sync_copy(zeros_hbm, accum)

    pltpu.sync_copy(dst_hbm.at[pl.ds(base, EPW)], idx_all)
    pltpu.sync_copy(ones_hbm, ones_v)
    plsc.subcore_barrier()

    def chunk(j, carry):
        off = pl.multiple_of(j * DCH, 8)
        pltpu.sync_copy(ones_v, accum.at[idx_all.at[pl.ds(off, DCH)]], add=True)
        return carry

    lax.fori_loop(0, NDCH, chunk, 0)
    plsc.subcore_barrier()

    @pl.when(s == 0)
    def _():
        pltpu.sync_copy(accum, degp_hbm.at[c])


_deg_kernel = pl.kernel(
    _deg_body,
    out_type=jax.ShapeDtypeStruct((NC, N), jnp.float32),
    mesh=_mesh,
    scratch_types=[
        pltpu.VMEM((EPW,), jnp.int32),
        pltpu.VMEM((DCH,), jnp.float32),
        pltpu.VMEM_SHARED((N,), jnp.float32),
        pltpu.SemaphoreType.DMA,
    ],
)


ZR = 624
ZR_LAST = N - 15 * ZR


def _seg_body(src_hbm, dst_hbm, hs_hbm, zeros_hbm, segp_hbm,
              sidx_all, didx_all, rows0, rows1, accum, sem0, sem1):
    c, s, wid = _ids()
    base = wid * EPW

    @pl.when(s < 15)
    def _():
        o = pl.multiple_of(s * ZR, 8)
        pltpu.sync_copy(zeros_hbm.at[pl.ds(o, ZR)], accum.at[pl.ds(o, ZR)])

    @pl.when(s == 15)
    def _():
        pltpu.sync_copy(zeros_hbm.at[pl.ds(15 * ZR, ZR_LAST)],
                        accum.at[pl.ds(15 * ZR, ZR_LAST)])

    pltpu.sync_copy(src_hbm.at[pl.ds(base, EPW)], sidx_all)
    pltpu.sync_copy(dst_hbm.at[pl.ds(base, EPW)], didx_all)
    plsc.subcore_barrier()

    def gstart(ch, rows, sem):
        o = pl.multiple_of(ch * CHUNK, 8)
        pltpu.async_copy(hs_hbm.at[sidx_all.at[pl.ds(o, CHUNK)]], rows, sem)

    def gwait(ch, rows, sem):
        o = pl.multiple_of(ch * CHUNK, 8)
        pltpu.make_async_copy(hs_hbm.at[sidx_all.at[pl.ds(o, CHUNK)]],
                              rows, sem).wait()

    def scat(ch, rows):
        o = pl.multiple_of(ch * CHUNK, 8)
        pltpu.sync_copy(rows, accum.at[didx_all.at[pl.ds(o, CHUNK)]], add=True)

    gstart(0, rows0, sem0)

    def pairstep(t, carry):
        c0, c1, c2 = 2 * t, 2 * t + 1, 2 * t + 2
        gstart(c1, rows1, sem1)
        gwait(c0, rows0, sem0)
        scat(c0, rows0)
        gstart(c2, rows0, sem0)
        gwait(c1, rows1, sem1)
        scat(c1, rows1)
        return carry

    lax.fori_loop(0, (NCHUNK - 1) // 2, pairstep, 0)
    gwait(NCHUNK - 1, rows0, sem0)
    scat(NCHUNK - 1, rows0)
    to = pl.multiple_of(NCHUNK * CHUNK, 8)
    tail_rows = rows1.at[pl.ds(0, TAIL)]
    pltpu.async_copy(hs_hbm.at[sidx_all.at[pl.ds(to, TAIL)]],
                     tail_rows, sem1).wait()
    pltpu.sync_copy(tail_rows, accum.at[didx_all.at[pl.ds(to, TAIL)]], add=True)
    plsc.subcore_barrier()

    @pl.when(s < 15)
    def _():
        o = pl.multiple_of(s * ZR, 8)
        pltpu.sync_copy(accum.at[pl.ds(o, ZR)], segp_hbm.at[c, pl.ds(o, ZR)])

    @pl.when(s == 15)
    def _():
        pltpu.sync_copy(accum.at[pl.ds(15 * ZR, ZR_LAST)],
                        segp_hbm.at[c, pl.ds(15 * ZR, ZR_LAST)])


_seg_kernel = pl.kernel(
    _seg_body,
    out_type=jax.ShapeDtypeStruct((NC, N, D), jnp.float32),
    mesh=_mesh,
    scratch_types=[
        pltpu.VMEM((EPW,), jnp.int32),
        pltpu.VMEM((EPW,), jnp.int32),
        pltpu.VMEM((CHUNK, D), jnp.float32),
        pltpu.VMEM((CHUNK, D), jnp.float32),
        pltpu.VMEM_SHARED((N, D), jnp.float32),
        pltpu.SemaphoreType.DMA,
        pltpu.SemaphoreType.DMA,
    ],
)


def _pair_body(src_hbm, dst_hbm, tab_hbm, xs_hbm, ys_hbm,
               sidx_all, didx_all, xr0, yr0, xr1, yr1,
               sx0, sy0, sx1, sy1):
    c, s, wid = _ids()
    base = wid * SEPW

    pltpu.sync_copy(src_hbm.at[pl.ds(base, SEPW)], sidx_all)
    pltpu.sync_copy(dst_hbm.at[pl.ds(base, SEPW)], didx_all)

    def gstart(ch, xr, yr, sx, sy):
        o = pl.multiple_of(ch * PCH, 8)
        pltpu.async_copy(tab_hbm.at[sidx_all.at[pl.ds(o, PCH)]], xr, sx)
        pltpu.async_copy(tab_hbm.at[didx_all.at[pl.ds(o, PCH)]], yr, sy)

    def finish(ch, xr, yr, sx, sy):
        o = pl.multiple_of(ch * PCH, 8)
        go = pl.multiple_of(base + ch * PCH, 8)
        pltpu.make_async_copy(tab_hbm.at[sidx_all.at[pl.ds(o, PCH)]], xr, sx).wait()
        pltpu.make_async_copy(tab_hbm.at[didx_all.at[pl.ds(o, PCH)]], yr, sy).wait()
        pltpu.sync_copy(xr, xs_hbm.at[pl.ds(go, PCH)])
        pltpu.sync_copy(yr, ys_hbm.at[pl.ds(go, PCH)])

    gstart(0, xr0, yr0, sx0, sy0)

    def pairstep(t, carry):
        c0, c1, c2 = 2 * t, 2 * t + 1, 2 * t + 2
        gstart(c1, xr1, yr1, sx1, sy1)
        finish(c0, xr0, yr0, sx0, sy0)
        gstart(c2, xr0, yr0, sx0, sy0)
        finish(c1, xr1, yr1, sx1, sy1)
        return carry

    lax.fori_loop(0, NPCH // 2 - 1, pairstep, 0)
    gstart(NPCH - 1, xr1, yr1, sx1, sy1)
    finish(NPCH - 2, xr0, yr0, sx0, sy0)
    finish(NPCH - 1, xr1, yr1, sx1, sy1)


_pair_kernel = pl.kernel(
    _pair_body,
    out_type=(jax.ShapeDtypeStruct((SE, D), jnp.float32),
              jax.ShapeDtypeStruct((SE, D), jnp.float32)),
    mesh=_mesh,
    scratch_types=[
        pltpu.VMEM((SEPW,), jnp.int32),
        pltpu.VMEM((SEPW,), jnp.int32),
        pltpu.VMEM((PCH, D), jnp.float32),
        pltpu.VMEM((PCH, D), jnp.float32),
        pltpu.VMEM((PCH, D), jnp.float32),
        pltpu.VMEM((PCH, D), jnp.float32),
        pltpu.SemaphoreType.DMA,
        pltpu.SemaphoreType.DMA,
        pltpu.SemaphoreType.DMA,
        pltpu.SemaphoreType.DMA,
    ],
)


NBLK = 1000
EBLK = 3200


def _dinv_block(degp):
    deg = 1.0 + degp[0] + degp[1]
    return lax.rsqrt(deg)


def _tc1_body(x_ref, w_ref, degp_ref, hs_ref):
    h = jnp.dot(x_ref[...], w_ref[...], preferred_element_type=jnp.float32)
    hs_ref[...] = _dinv_block(degp_ref[...]) * h


def _tc2_body(segp_ref, hs_ref, degp_ref, b_ref, w_ref, out_ref):
    dinv = _dinv_block(degp_ref[...])
    seg = segp_ref[0] + segp_ref[1] + hs_ref[...]
    o = jax.nn.relu(dinv * seg + b_ref[...])
    h = jnp.dot(o, w_ref[...], preferred_element_type=jnp.float32)
    out_ref[...] = dinv * h


def _tc3_body(segp_ref, hs_ref, degp_ref, b_ref, out_ref):
    dinv = _dinv_block(degp_ref[...])
    seg = segp_ref[0] + segp_ref[1] + hs_ref[...]
    out_ref[...] = jax.nn.relu(dinv * seg + b_ref[...])


def _head_body(xs_ref, ys_ref, fa_ref, fb_ref, b1_ref, w2_ref, b2_ref, out_ref):
    x = xs_ref[...]
    y = ys_ref[...]
    h = jnp.dot(x * y, fa_ref[...], preferred_element_type=jnp.float32)
    h += jnp.dot(x - y, fb_ref[...], preferred_element_type=jnp.float32)
    h = jax.nn.relu(h + b1_ref[...])
    p = jnp.dot(h, w2_ref[...], preferred_element_type=jnp.float32)
    out_ref[...] = jax.nn.sigmoid(p + b2_ref[...])


def _node_spec(d):
    return pl.BlockSpec((NBLK, d), lambda i: (i, 0))


_full = lambda shape: pl.BlockSpec(shape, lambda i: tuple(0 for _ in shape))
_degp_spec = pl.BlockSpec((NC, NBLK, 1), lambda i: (0, i, 0))
_segp_spec = pl.BlockSpec((NC, NBLK, D), lambda i: (0, i, 0))

_tc1 = pl.pallas_call(
    _tc1_body,
    grid=(N // NBLK,),
    in_specs=[_node_spec(D), _full((D, D)), _degp_spec],
    out_specs=_node_spec(D),
    out_shape=jax.ShapeDtypeStruct((N, D), jnp.float32),
)

_tc2 = pl.pallas_call(
    _tc2_body,
    grid=(N // NBLK,),
    in_specs=[_segp_spec, _node_spec(D), _degp_spec, _full((1, D)), _full((D, D))],
    out_specs=_node_spec(D),
    out_shape=jax.ShapeDtypeStruct((N, D), jnp.float32),
)

_tc3 = pl.pallas_call(
    _tc3_body,
    grid=(N // NBLK,),
    in_specs=[_segp_spec, _node_spec(D), _degp_spec, _full((1, D))],
    out_specs=_node_spec(D),
    out_shape=jax.ShapeDtypeStruct((N, D), jnp.float32),
)

_head = pl.pallas_call(
    _head_body,
    grid=(SE // EBLK,),
    in_specs=[
        pl.BlockSpec((EBLK, D), lambda i: (i, 0)),
        pl.BlockSpec((EBLK, D), lambda i: (i, 0)),
        _full((D, D)), _full((D, D)), _full((1, D)),
        _full((D, 1)), _full((1, 1)),
    ],
    out_specs=pl.BlockSpec((EBLK, 1), lambda i: (i, 0)),
    out_shape=jax.ShapeDtypeStruct((SE, 1), jnp.float32),
)


def kernel(node_features, W1, b1, W2, b2, fc1_w, fc1_b, fc2_w, fc2_b, edge_index):
    src = edge_index[0].astype(jnp.int32)
    dst = edge_index[1].astype(jnp.int32)
    ones1 = jnp.ones((DCH,), jnp.float32)
    zeros1 = jnp.zeros((N,), jnp.float32)
    zerosD = jnp.zeros((N, D), jnp.float32)
    b1r = b1.reshape(1, D)
    b2r = b2.reshape(1, D)
    fc1_br = fc1_b.reshape(1, D)
    fc2_br = fc2_b.reshape(1, 1)
    fa = fc1_w[:D]
    fb = fc1_w[D:]

    degp = _deg_kernel(dst, ones1, zeros1).reshape(NC, N, 1)
    hs1 = _tc1(node_features, W1, degp)
    segp1 = _seg_kernel(src, dst, hs1, zerosD)
    hs2 = _tc2(segp1, hs1, degp, b1r, W2)
    segp2 = _seg_kernel(src, dst, hs2, zerosD)
    out2 = _tc3(segp2, hs2, degp, b2r)
    probs = []
    for i in range(NSLICE):
        xs, ys = _pair_kernel(src[i * SE:(i + 1) * SE],
                              dst[i * SE:(i + 1) * SE], out2)
        probs.append(_head(xs, ys, fa, fb, fc1_br, fc2_w, fc2_br))
    return jnp.concatenate(probs, axis=0)


# speedup vs baseline: 1.1723x; 1.0155x over previous
import functools

import jax
import jax.numpy as jnp
from jax import lax
from jax.experimental import pallas as pl
from jax.experimental.pallas import tpu as pltpu
from jax.experimental.pallas import tpu_sc as plsc

N = 10000
E = 320000
D = 128

NC = 2
NS = 16
NW = NC * NS
EPW = E // NW
CHUNK = 112
NCHUNK = 89
TAIL = EPW - NCHUNK * CHUNK
NSLICE = 5
SE = E // NSLICE
SEPW = SE // NW
PCH = 200
NPCH = SEPW // PCH
DCH = 2000
NDCH = EPW // DCH

_mesh = plsc.VectorSubcoreMesh(core_axis_name="c", subcore_axis_name="s")


def _ids():
    c = lax.axis_index("c")
    s = lax.axis_index("s")
    wid = s * NC + c
    return c, s, wid


def _deg_body(dst_hbm, ones_hbm, zeros_hbm, degp_hbm, idx_all, ones_v, accum, sem):
    c, s, wid = _ids()
    base = wid * EPW

    @pl.when(s == 0)
    def _():
        pltpu.sync_copy(zeros_hbm, accum)

    pltpu.sync_copy(dst_hbm.at[pl.ds(base, EPW)], idx_all)
    pltpu.sync_copy(ones_hbm, ones_v)
    plsc.subcore_barrier()

    def chunk(j, carry):
        off = pl.multiple_of(j * DCH, 8)
        pltpu.sync_copy(ones_v, accum.at[idx_all.at[pl.ds(off, DCH)]], add=True)
        return carry

    lax.fori_loop(0, NDCH, chunk, 0)
    plsc.subcore_barrier()

    @pl.when(s == 0)
    def _():
        pltpu.sync_copy(accum, degp_hbm.at[c])


_deg_kernel = pl.kernel(
    _deg_body,
    out_type=jax.ShapeDtypeStruct((NC, N), jnp.float32),
    mesh=_mesh,
    scratch_types=[
        pltpu.VMEM((EPW,), jnp.int32),
        pltpu.VMEM((DCH,), jnp.float32),
        pltpu.VMEM_SHARED((N,), jnp.float32),
        pltpu.SemaphoreType.DMA,
    ],
)


ZR = 624
ZR_LAST = N - 15 * ZR


def _seg_body(src_hbm, dst_hbm, hs_hbm, zeros_hbm, segp_hbm,
              sidx_all, didx_all, rows0, rows1, accum, sem0, sem1):
    c, s, wid = _ids()
    base = wid * EPW

    @pl.when(s < 15)
    def _():
        o = pl.multiple_of(s * ZR, 8)
        pltpu.sync_copy(zeros_hbm.at[pl.ds(o, ZR)], accum.at[pl.ds(o, ZR)])

    @pl.when(s == 15)
    def _():
        pltpu.sync_copy(zeros_hbm.at[pl.ds(15 * ZR, ZR_LAST)],
                        accum.at[pl.ds(15 * ZR, ZR_LAST)])

    pltpu.sync_copy(src_hbm.at[pl.ds(base, EPW)], sidx_all)
    pltpu.sync_copy(dst_hbm.at[pl.ds(base, EPW)], didx_all)
    plsc.subcore_barrier()

    def gstart(ch, rows, sem):
        o = pl.multiple_of(ch * CHUNK, 8)
        pltpu.async_copy(hs_hbm.at[sidx_all.at[pl.ds(o, CHUNK)]], rows, sem)

    def gwait(ch, rows, sem):
        o = pl.multiple_of(ch * CHUNK, 8)
        pltpu.make_async_copy(hs_hbm.at[sidx_all.at[pl.ds(o, CHUNK)]],
                              rows, sem).wait()

    def scat(ch, rows):
        o = pl.multiple_of(ch * CHUNK, 8)
        pltpu.sync_copy(rows, accum.at[didx_all.at[pl.ds(o, CHUNK)]], add=True)

    gstart(0, rows0, sem0)

    def pairstep(t, carry):
        c0, c1, c2 = 2 * t, 2 * t + 1, 2 * t + 2
        gstart(c1, rows1, sem1)
        gwait(c0, rows0, sem0)
        scat(c0, rows0)
        gstart(c2, rows0, sem0)
        gwait(c1, rows1, sem1)
        scat(c1, rows1)
        return carry

    lax.fori_loop(0, (NCHUNK - 1) // 2, pairstep, 0)
    gwait(NCHUNK - 1, rows0, sem0)
    scat(NCHUNK - 1, rows0)
    to = pl.multiple_of(NCHUNK * CHUNK, 8)
    tail_rows = rows1.at[pl.ds(0, TAIL)]
    pltpu.async_copy(hs_hbm.at[sidx_all.at[pl.ds(to, TAIL)]],
                     tail_rows, sem1).wait()
    pltpu.sync_copy(tail_rows, accum.at[didx_all.at[pl.ds(to, TAIL)]], add=True)
    plsc.subcore_barrier()

    @pl.when(s < 15)
    def _():
        o = pl.multiple_of(s * ZR, 8)
        pltpu.sync_copy(accum.at[pl.ds(o, ZR)], segp_hbm.at[c, pl.ds(o, ZR)])

    @pl.when(s == 15)
    def _():
        pltpu.sync_copy(accum.at[pl.ds(15 * ZR, ZR_LAST)],
                        segp_hbm.at[c, pl.ds(15 * ZR, ZR_LAST)])


_seg_kernel = pl.kernel(
    _seg_body,
    out_type=jax.ShapeDtypeStruct((NC, N, D), jnp.float32),
    mesh=_mesh,
    scratch_types=[
        pltpu.VMEM((EPW,), jnp.int32),
        pltpu.VMEM((EPW,), jnp.int32),
        pltpu.VMEM((CHUNK, D), jnp.float32),
        pltpu.VMEM((CHUNK, D), jnp.float32),
        pltpu.VMEM_SHARED((N, D), jnp.float32),
        pltpu.SemaphoreType.DMA,
        pltpu.SemaphoreType.DMA,
    ],
)


def _pair_body(src_hbm, dst_hbm, tab_hbm, xs_hbm, ys_hbm,
               sidx_all, didx_all, xr0, yr0, xr1, yr1,
               sx0, sy0, sx1, sy1):
    c, s, wid = _ids()
    base = wid * SEPW

    pltpu.sync_copy(src_hbm.at[pl.ds(base, SEPW)], sidx_all)
    pltpu.sync_copy(dst_hbm.at[pl.ds(base, SEPW)], didx_all)

    def gstart(ch, xr, yr, sx, sy):
        o = pl.multiple_of(ch * PCH, 8)
        pltpu.async_copy(tab_hbm.at[sidx_all.at[pl.ds(o, PCH)]], xr, sx)
        pltpu.async_copy(tab_hbm.at[didx_all.at[pl.ds(o, PCH)]], yr, sy)

    def finish(ch, xr, yr, sx, sy):
        o = pl.multiple_of(ch * PCH, 8)
        go = pl.multiple_of(base + ch * PCH, 8)
        pltpu.make_async_copy(tab_hbm.at[sidx_all.at[pl.ds(o, PCH)]], xr, sx).wait()
        pltpu.make_async_copy(tab_hbm.at[didx_all.at[pl.ds(o, PCH)]], yr, sy).wait()
        pltpu.sync_copy(xr, xs_hbm.at[pl.ds(go, PCH)])
        pltpu.sync_copy(yr, ys_hbm.at[pl.ds(go, PCH)])

    gstart(0, xr0, yr0, sx0, sy0)

    def pairstep(t, carry):
        c0, c1, c2 = 2 * t, 2 * t + 1, 2 * t + 2
        gstart(c1, xr1, yr1, sx1, sy1)
        finish(c0, xr0, yr0, sx0, sy0)
        gstart(c2, xr0, yr0, sx0, sy0)
        finish(c1, xr1, yr1, sx1, sy1)
        return carry

    lax.fori_loop(0, NPCH // 2 - 1, pairstep, 0)
    gstart(NPCH - 1, xr1, yr1, sx1, sy1)
    finish(NPCH - 2, xr0, yr0, sx0, sy0)
    finish(NPCH - 1, xr1, yr1, sx1, sy1)


_pair_kernel = pl.kernel(
    _pair_body,
    out_type=(jax.ShapeDtypeStruct((SE, D), jnp.float32),
              jax.ShapeDtypeStruct((SE, D), jnp.float32)),
    mesh=_mesh,
    scratch_types=[
        pltpu.VMEM((SEPW,), jnp.int32),
        pltpu.VMEM((SEPW,), jnp.int32),
        pltpu.VMEM((PCH, D), jnp.float32),
        pltpu.VMEM((PCH, D), jnp.float32),
        pltpu.VMEM((PCH, D), jnp.float32),
        pltpu.VMEM((PCH, D), jnp.float32),
        pltpu.SemaphoreType.DMA,
        pltpu.SemaphoreType.DMA,
        pltpu.SemaphoreType.DMA,
        pltpu.SemaphoreType.DMA,
    ],
)


NBLK = 1000
EBLK = 6400


def _dinv_block(degp):
    deg = 1.0 + degp[0] + degp[1]
    return lax.rsqrt(deg)


def _tc1_body(x_ref, w_ref, degp_ref, hs_ref):
    h = jnp.dot(x_ref[...], w_ref[...], preferred_element_type=jnp.float32)
    hs_ref[...] = _dinv_block(degp_ref[...]) * h


def _tc2_body(segp_ref, hs_ref, degp_ref, b_ref, w_ref, out_ref):
    dinv = _dinv_block(degp_ref[...])
    seg = segp_ref[0] + segp_ref[1] + hs_ref[...]
    o = jax.nn.relu(dinv * seg + b_ref[...])
    h = jnp.dot(o, w_ref[...], preferred_element_type=jnp.float32)
    out_ref[...] = dinv * h


def _tc3_body(segp_ref, hs_ref, degp_ref, b_ref, out_ref):
    dinv = _dinv_block(degp_ref[...])
    seg = segp_ref[0] + segp_ref[1] + hs_ref[...]
    out_ref[...] = jax.nn.relu(dinv * seg + b_ref[...])


def _head_body(xs_ref, ys_ref, fa_ref, fb_ref, b1_ref, w2_ref, b2_ref, out_ref):
    x = xs_ref[...]
    y = ys_ref[...]
    h = jnp.dot(x * y, fa_ref[...], preferred_element_type=jnp.float32)
    h += jnp.dot(x - y, fb_ref[...], preferred_element_type=jnp.float32)
    h = jax.nn.relu(h + b1_ref[...])
    p = jnp.dot(h, w2_ref[...], preferred_element_type=jnp.float32)
    out_ref[...] = jax.nn.sigmoid(p + b2_ref[...])


def _node_spec(d):
    return pl.BlockSpec((NBLK, d), lambda i: (i, 0))


_full = lambda shape: pl.BlockSpec(shape, lambda i: tuple(0 for _ in shape))
_degp_spec = pl.BlockSpec((NC, NBLK, 1), lambda i: (0, i, 0))
_segp_spec = pl.BlockSpec((NC, NBLK, D), lambda i: (0, i, 0))

_tc1 = pl.pallas_call(
    _tc1_body,
    grid=(N // NBLK,),
    in_specs=[_node_spec(D), _full((D, D)), _degp_spec],
    out_specs=_node_spec(D),
    out_shape=jax.ShapeDtypeStruct((N, D), jnp.float32),
)

_tc2 = pl.pallas_call(
    _tc2_body,
    grid=(N // NBLK,),
    in_specs=[_segp_spec, _node_spec(D), _degp_spec, _full((1, D)), _full((D, D))],
    out_specs=_node_spec(D),
    out_shape=jax.ShapeDtypeStruct((N, D), jnp.float32),
)

_tc3 = pl.pallas_call(
    _tc3_body,
    grid=(N // NBLK,),
    in_specs=[_segp_spec, _node_spec(D), _degp_spec, _full((1, D))],
    out_specs=_node_spec(D),
    out_shape=jax.ShapeDtypeStruct((N, D), jnp.float32),
)

_head = pl.pallas_call(
    _head_body,
    grid=(SE // EBLK,),
    in_specs=[
        pl.BlockSpec((EBLK, D), lambda i: (i, 0)),
        pl.BlockSpec((EBLK, D), lambda i: (i, 0)),
        _full((D, D)), _full((D, D)), _full((1, D)),
        _full((D, 1)), _full((1, 1)),
    ],
    out_specs=pl.BlockSpec((EBLK, 1), lambda i: (i, 0)),
    out_shape=jax.ShapeDtypeStruct((SE, 1), jnp.float32),
)


def kernel(node_features, W1, b1, W2, b2, fc1_w, fc1_b, fc2_w, fc2_b, edge_index):
    src = edge_index[0].astype(jnp.int32)
    dst = edge_index[1].astype(jnp.int32)
    ones1 = jnp.ones((DCH,), jnp.float32)
    zeros1 = jnp.zeros((N,), jnp.float32)
    zerosD = jnp.zeros((N, D), jnp.float32)
    b1r = b1.reshape(1, D)
    b2r = b2.reshape(1, D)
    fc1_br = fc1_b.reshape(1, D)
    fc2_br = fc2_b.reshape(1, 1)
    fa = fc1_w[:D]
    fb = fc1_w[D:]

    degp = _deg_kernel(dst, ones1, zeros1).reshape(NC, N, 1)
    hs1 = _tc1(node_features, W1, degp)
    segp1 = _seg_kernel(src, dst, hs1, zerosD)
    hs2 = _tc2(segp1, hs1, degp, b1r, W2)
    segp2 = _seg_kernel(src, dst, hs2, zerosD)
    out2 = _tc3(segp2, hs2, degp, b2r)
    probs = []
    for i in range(NSLICE):
        xs, ys = _pair_kernel(src[i * SE:(i + 1) * SE],
                              dst[i * SE:(i + 1) * SE], out2)
        probs.append(_head(xs, ys, fa, fb, fc1_br, fc2_w, fc2_br))
    return jnp.concatenate(probs, axis=0)
